# baseline (speedup 1.0000x reference)
import functools

import jax
import jax.numpy as jnp
from jax import lax
from jax.experimental import pallas as pl
from jax.experimental.pallas import tpu as pltpu
from jax.experimental.pallas import tpu_sc as plsc

B = 16384
NV = 100
NG = 1000
NW = 32
RW = B // NW
CH = 64
NCH = RW // CH
L = 16
NF = 6
TB = NV - L


def _sc_body(gene_hbm, table_hbm, out_hbm,
             table_v, g0_v, g1_v, stage_v, sem_t, sem0, sem1):
    wid = lax.axis_index("s") * 2 + lax.axis_index("c")
    base_row = wid * RW

    tbl_cp = pltpu.make_async_copy(table_hbm, table_v, sem_t)
    tbl_cp.start()

    bufs = (g0_v, g1_v)
    sems = (sem0, sem1)

    def gene_copy(c):
        return pltpu.make_async_copy(
            gene_hbm.at[pl.ds(base_row + c * CH, CH), :],
            bufs[c % 2], sems[c % 2])

    cp = gene_copy(0)
    cp.start()
    tbl_cp.wait()

    lanes = lax.iota(jnp.int32, L)
    tail_fresh = lanes >= (L - (NV - NF * L))
    vids = [lanes + k * L for k in range(NF)]
    vids_tail = lanes + TB
    fzero = jnp.zeros((L,), jnp.float32)
    izero = jnp.zeros((L,), jnp.int32)

    for c in range(NCH):
        cp.wait()
        if c + 1 < NCH:
            cp = gene_copy(c + 1)
            cp.start()
        gbuf = bufs[c % 2]

        def rows_body(i, _):
            rbase = i * L
            sums = fzero
            for j in range(L):
                r = rbase + j
                acc = fzero
                for k in range(NF):
                    g = gbuf[r, pl.ds(k * L, L)]
                    acc = acc + plsc.load_gather(table_v, [vids[k], g])
                rvec = jnp.full((L,), r, jnp.int32)
                gt = plsc.load_gather(gbuf, [rvec, TB + lanes])
                vt = plsc.load_gather(table_v, [vids_tail, gt])
                acc = acc + jnp.where(tail_fresh, vt, 0.0)
                sums = jnp.where(lanes == j, jnp.sum(acc), sums)
            plsc.store_scatter(stage_v, [lanes, izero], sums)
            out_row = base_row + c * CH + rbase
            pltpu.sync_copy(stage_v, out_hbm.at[pl.ds(out_row, L), :])
            return 0

        lax.fori_loop(0, CH // L, rows_body, 0)


@jax.jit
def kernel(gene, genes):
    table2d = jnp.squeeze(genes, -1)

    sc_call = functools.partial(
        pl.kernel,
        mesh=plsc.VectorSubcoreMesh(core_axis_name="c", subcore_axis_name="s"),
        out_type=jax.ShapeDtypeStruct((B, 1), jnp.float32),
        scratch_types=[
            pltpu.VMEM((NV, NG), jnp.float32),
            pltpu.VMEM((CH, NV), jnp.int32),
            pltpu.VMEM((CH, NV), jnp.int32),
            pltpu.VMEM((L, 1), jnp.float32),
            pltpu.SemaphoreType.DMA,
            pltpu.SemaphoreType.DMA,
            pltpu.SemaphoreType.DMA,
        ],
        compiler_params=pltpu.CompilerParams(needs_layout_passes=False),
    )(_sc_body)

    return sc_call(gene.astype(jnp.int32), table2d)


# speedup vs baseline: 1.5955x; 1.5955x over previous
import functools

import jax
import jax.numpy as jnp
from jax import lax
from jax.experimental import pallas as pl
from jax.experimental.pallas import tpu as pltpu
from jax.experimental.pallas import tpu_sc as plsc

B = 16384
NV = 100
NG = 1000
NW = 32
RW = B // NW
CH = 128
NCH = RW // CH
L = 16


def _sc_body(gene_hbm, table_hbm, out_hbm,
             table_v, g0_v, g1_v, out_v, sem_t, sem0, sem1):
    wid = lax.axis_index("s") * 2 + lax.axis_index("c")
    base_row = wid * RW

    tbl_cp = pltpu.make_async_copy(table_hbm, table_v, sem_t)
    tbl_cp.start()

    bufs = (g0_v, g1_v)
    sems = (sem0, sem1)

    def gene_copy(c):
        return pltpu.make_async_copy(
            gene_hbm.at[:, pl.ds(base_row + c * CH, CH)],
            bufs[c % 2], sems[c % 2])

    cp = gene_copy(0)
    cp.start()
    tbl_cp.wait()

    fzero = jnp.zeros((L,), jnp.float32)

    for c in range(NCH):
        cp.wait()
        if c + 1 < NCH:
            cp = gene_copy(c + 1)
            cp.start()
        gbuf = bufs[c % 2]

        def group_body(i0, _):
            col = i0 * L

            def v_body(v, acc):
                g = gbuf[v, pl.ds(col, L)]
                return acc + plsc.load_gather(table_v, [g + v * NG])

            acc = lax.fori_loop(0, NV, v_body, fzero, unroll=10)
            out_v[pl.ds(c * CH + col, L)] = acc
            return 0

        lax.fori_loop(0, CH // L, group_body, 0)

    pltpu.sync_copy(out_v, out_hbm.at[pl.ds(base_row, RW)])


@jax.jit
def kernel(gene, genes):
    gene_t = gene.astype(jnp.int32).T
    table_flat = genes.reshape(-1).astype(jnp.float32)

    sc_call = functools.partial(
        pl.kernel,
        mesh=plsc.VectorSubcoreMesh(core_axis_name="c", subcore_axis_name="s"),
        out_type=jax.ShapeDtypeStruct((B,), jnp.float32),
        scratch_types=[
            pltpu.VMEM((NV * NG,), jnp.float32),
            pltpu.VMEM((NV, CH), jnp.int32),
            pltpu.VMEM((NV, CH), jnp.int32),
            pltpu.VMEM((RW,), jnp.float32),
            pltpu.SemaphoreType.DMA,
            pltpu.SemaphoreType.DMA,
            pltpu.SemaphoreType.DMA,
        ],
        compiler_params=pltpu.CompilerParams(needs_layout_passes=False),
    )(_sc_body)

    fit = sc_call(gene_t, table_flat)
    return fit.reshape(B, 1)
